# speedup vs baseline: 3.0154x; 3.0154x over previous
import functools

import jax
import jax.numpy as jnp
from jax import lax
from jax.experimental import pallas as pl
from jax.experimental.pallas import tpu as pltpu
from jax.experimental.pallas import tpu_sc as plsc

_VOCAB = 100000
_HIDDEN = 768
_SEQ = 512
_BSZ = 128
_EPS = 1e-6

_B = _BSZ * _SEQ
_NC = 2
_NS = 16
_NW = _NC * _NS
_B_PER_W = _B // _NW
_CHUNK = 128
_N_CHUNKS = _B_PER_W // _CHUNK


def _sc_gather_body(table_hbm, idx_hbm, out_hbm, idx_v, rows_v, sem):
    wid = lax.axis_index("s") * _NC + lax.axis_index("c")
    base = wid * _B_PER_W

    def chunk(ci, _):
        off = base + ci * _CHUNK
        pltpu.sync_copy(idx_hbm.at[pl.ds(off, _CHUNK)], idx_v)
        pltpu.async_copy(table_hbm.at[idx_v], rows_v, sem).wait()
        pltpu.sync_copy(rows_v, out_hbm.at[pl.ds(off, _CHUNK)])
        return ()

    lax.fori_loop(0, _N_CHUNKS, chunk, ())


_sc_gather = functools.partial(
    pl.kernel,
    mesh=plsc.VectorSubcoreMesh(core_axis_name="c", subcore_axis_name="s"),
    out_type=jax.ShapeDtypeStruct((_B, _HIDDEN), jnp.float32),
    scratch_types=[
        pltpu.VMEM((_CHUNK,), jnp.int32),
        pltpu.VMEM((_CHUNK, _HIDDEN), jnp.float32),
        pltpu.SemaphoreType.DMA,
    ],
)(_sc_gather_body)


def _tc_body(x_ref, tt_ref, pos_ref, type_ref, gamma_ref, beta_ref, o_ref):
    x = x_ref[0]
    t = tt_ref[0, 0].astype(jnp.float32)
    pos = pos_ref[...]
    t0 = type_ref[0]
    dt = type_ref[1] - type_ref[0]
    x = x + pos + t0[None, :] + t[:, None] * dt[None, :]
    mean = jnp.mean(x, axis=-1, keepdims=True)
    xc = x - mean
    var = jnp.mean(xc * xc, axis=-1, keepdims=True)
    y = xc * lax.rsqrt(var + _EPS)
    o_ref[0] = y * gamma_ref[0][None, :] + beta_ref[0][None, :]


def _tc_addln(gathered, tt, w_pos, w_type8, gamma2d, beta2d):
    return pl.pallas_call(
        _tc_body,
        grid=(_BSZ,),
        in_specs=[
            pl.BlockSpec((1, _SEQ, _HIDDEN), lambda b: (b, 0, 0)),
            pl.BlockSpec((1, 1, _SEQ), lambda b: (b, 0, 0)),
            pl.BlockSpec((_SEQ, _HIDDEN), lambda b: (0, 0)),
            pl.BlockSpec((8, _HIDDEN), lambda b: (0, 0)),
            pl.BlockSpec((1, _HIDDEN), lambda b: (0, 0)),
            pl.BlockSpec((1, _HIDDEN), lambda b: (0, 0)),
        ],
        out_specs=pl.BlockSpec((1, _SEQ, _HIDDEN), lambda b: (b, 0, 0)),
        out_shape=jax.ShapeDtypeStruct((_BSZ, _SEQ, _HIDDEN), jnp.float32),
    )(gathered, tt, w_pos, w_type8, gamma2d, beta2d)


def kernel(input_ids, token_type_ids, W_word, W_pos, W_type, gamma, beta):
    idx_flat = input_ids.reshape(-1).astype(jnp.int32)
    gathered = _sc_gather(W_word, idx_flat)
    gathered = gathered.reshape(_BSZ, _SEQ, _HIDDEN)
    tt = token_type_ids.astype(jnp.int32).reshape(_BSZ, 1, _SEQ)
    w_type8 = jnp.zeros((8, _HIDDEN), jnp.float32).at[:2].set(W_type)
    return _tc_addln(gathered, tt, W_pos, w_type8,
                     gamma.reshape(1, _HIDDEN), beta.reshape(1, _HIDDEN))


# speedup vs baseline: 3.0333x; 1.0059x over previous
import functools

import jax
import jax.numpy as jnp
from jax import lax
from jax.experimental import pallas as pl
from jax.experimental.pallas import tpu as pltpu
from jax.experimental.pallas import tpu_sc as plsc

_VOCAB = 100000
_HIDDEN = 768
_SEQ = 512
_BSZ = 128
_EPS = 1e-6

_B = _BSZ * _SEQ
_NC = 2
_NS = 16
_NW = _NC * _NS
_B_PER_W = _B // _NW
_CHUNK = 64
_N_CHUNKS = _B_PER_W // _CHUNK


def _sc_gather_body(table_hbm, idx_hbm, out_hbm, idx_v, rows_v, sem0, sem1):
    wid = lax.axis_index("s") * _NC + lax.axis_index("c")
    base = wid * _B_PER_W
    pltpu.sync_copy(idx_hbm.at[pl.ds(base, _B_PER_W)], idx_v)
    sems = (sem0, sem1)

    def start(ci, b):
        pltpu.async_copy(
            table_hbm.at[idx_v.at[pl.ds(ci * _CHUNK, _CHUNK)]],
            rows_v.at[b], sems[b])

    start(0, 0)
    start(1, 1)
    for ci in range(_N_CHUNKS):
        b = ci % 2
        pltpu.make_async_copy(
            table_hbm.at[idx_v.at[pl.ds(ci * _CHUNK, _CHUNK)]],
            rows_v.at[b], sems[b]).wait()
        pltpu.sync_copy(rows_v.at[b],
                        out_hbm.at[pl.ds(base + ci * _CHUNK, _CHUNK)])
        if ci + 2 < _N_CHUNKS:
            start(ci + 2, b)


_sc_gather = functools.partial(
    pl.kernel,
    mesh=plsc.VectorSubcoreMesh(core_axis_name="c", subcore_axis_name="s"),
    out_type=jax.ShapeDtypeStruct((_B, _HIDDEN), jnp.float32),
    scratch_types=[
        pltpu.VMEM((_B_PER_W,), jnp.int32),
        pltpu.VMEM((2, _CHUNK, _HIDDEN), jnp.float32),
        pltpu.SemaphoreType.DMA,
        pltpu.SemaphoreType.DMA,
    ],
)(_sc_gather_body)


def _tc_body(x_ref, tt_ref, pos_ref, type_ref, gamma_ref, beta_ref, o_ref):
    x = x_ref[0]
    t = tt_ref[0, 0].astype(jnp.float32)
    pos = pos_ref[...]
    t0 = type_ref[0]
    dt = type_ref[1] - type_ref[0]
    x = x + pos + t0[None, :] + t[:, None] * dt[None, :]
    mean = jnp.mean(x, axis=-1, keepdims=True)
    xc = x - mean
    var = jnp.mean(xc * xc, axis=-1, keepdims=True)
    y = xc * lax.rsqrt(var + _EPS)
    o_ref[0] = y * gamma_ref[0][None, :] + beta_ref[0][None, :]


def _tc_addln(gathered, tt, w_pos, w_type8, gamma2d, beta2d):
    return pl.pallas_call(
        _tc_body,
        grid=(_BSZ,),
        in_specs=[
            pl.BlockSpec((1, _SEQ, _HIDDEN), lambda b: (b, 0, 0)),
            pl.BlockSpec((1, 1, _SEQ), lambda b: (b, 0, 0)),
            pl.BlockSpec((_SEQ, _HIDDEN), lambda b: (0, 0)),
            pl.BlockSpec((8, _HIDDEN), lambda b: (0, 0)),
            pl.BlockSpec((1, _HIDDEN), lambda b: (0, 0)),
            pl.BlockSpec((1, _HIDDEN), lambda b: (0, 0)),
        ],
        out_specs=pl.BlockSpec((1, _SEQ, _HIDDEN), lambda b: (b, 0, 0)),
        out_shape=jax.ShapeDtypeStruct((_BSZ, _SEQ, _HIDDEN), jnp.float32),
    )(gathered, tt, w_pos, w_type8, gamma2d, beta2d)


def kernel(input_ids, token_type_ids, W_word, W_pos, W_type, gamma, beta):
    idx_flat = input_ids.reshape(-1).astype(jnp.int32)
    gathered = _sc_gather(W_word, idx_flat)
    gathered = gathered.reshape(_BSZ, _SEQ, _HIDDEN)
    tt = token_type_ids.astype(jnp.int32).reshape(_BSZ, 1, _SEQ)
    w_type8 = jnp.zeros((8, _HIDDEN), jnp.float32).at[:2].set(W_type)
    return _tc_addln(gathered, tt, W_pos, w_type8,
                     gamma.reshape(1, _HIDDEN), beta.reshape(1, _HIDDEN))


# speedup vs baseline: 3.3046x; 1.0894x over previous
import functools

import jax
import jax.numpy as jnp
from jax import lax
from jax.experimental import pallas as pl
from jax.experimental.pallas import tpu as pltpu
from jax.experimental.pallas import tpu_sc as plsc

_VOCAB = 100000
_HIDDEN = 768
_SEQ = 512
_BSZ = 128
_EPS = 1e-6

_B = _BSZ * _SEQ
_NC = 2
_NS = 16
_NW = _NC * _NS
_CHUNK = 64

_K = 4
_BK = _BSZ // _K
_TOK_K = _B // _K
_B_PER_W = _TOK_K // _NW
_N_CHUNKS = _B_PER_W // _CHUNK


def _sc_gather_body(table_hbm, idx_hbm, out_hbm, idx_v, rows_v, sem0, sem1):
    wid = lax.axis_index("s") * _NC + lax.axis_index("c")
    base = wid * _B_PER_W
    pltpu.sync_copy(idx_hbm.at[pl.ds(base, _B_PER_W)], idx_v)
    sems = (sem0, sem1)

    def start(ci, b):
        pltpu.async_copy(
            table_hbm.at[idx_v.at[pl.ds(ci * _CHUNK, _CHUNK)]],
            rows_v.at[b], sems[b])

    start(0, 0)
    start(1, 1)
    for ci in range(_N_CHUNKS):
        b = ci % 2
        pltpu.make_async_copy(
            table_hbm.at[idx_v.at[pl.ds(ci * _CHUNK, _CHUNK)]],
            rows_v.at[b], sems[b]).wait()
        pltpu.sync_copy(rows_v.at[b],
                        out_hbm.at[pl.ds(base + ci * _CHUNK, _CHUNK)])
        if ci + 2 < _N_CHUNKS:
            start(ci + 2, b)


_sc_gather = functools.partial(
    pl.kernel,
    mesh=plsc.VectorSubcoreMesh(core_axis_name="c", subcore_axis_name="s"),
    out_type=jax.ShapeDtypeStruct((_TOK_K, _HIDDEN), jnp.float32),
    scratch_types=[
        pltpu.VMEM((_B_PER_W,), jnp.int32),
        pltpu.VMEM((2, _CHUNK, _HIDDEN), jnp.float32),
        pltpu.SemaphoreType.DMA,
        pltpu.SemaphoreType.DMA,
    ],
)(_sc_gather_body)


def _tc_body(x_ref, tt_ref, pos_ref, type_ref, gamma_ref, beta_ref, *rest):
    o_ref = rest[-1]
    x = x_ref[0]
    t = tt_ref[0, 0].astype(jnp.float32)
    pos = pos_ref[...]
    t0 = type_ref[0]
    dt = type_ref[1] - type_ref[0]
    x = x + pos + t0[None, :] + t[:, None] * dt[None, :]
    mean = jnp.mean(x, axis=-1, keepdims=True)
    xc = x - mean
    var = jnp.mean(xc * xc, axis=-1, keepdims=True)
    y = xc * lax.rsqrt(var + _EPS)
    o_ref[0] = y * gamma_ref[0][None, :] + beta_ref[0][None, :]


def _tc_part(k, gathered_k, tt, w_pos, w_type8, gamma2d, beta2d, y_prev):
    ins = [gathered_k, tt, w_pos, w_type8, gamma2d, beta2d]
    in_specs = [
        pl.BlockSpec((1, _SEQ, _HIDDEN), lambda b: (b, 0, 0)),
        pl.BlockSpec((1, 1, _SEQ), lambda b, k=k: (b + k * _BK, 0, 0)),
        pl.BlockSpec((_SEQ, _HIDDEN), lambda b: (0, 0)),
        pl.BlockSpec((8, _HIDDEN), lambda b: (0, 0)),
        pl.BlockSpec((1, _HIDDEN), lambda b: (0, 0)),
        pl.BlockSpec((1, _HIDDEN), lambda b: (0, 0)),
    ]
    io_alias = {}
    if y_prev is not None:
        ins.append(y_prev)
        in_specs.append(pl.BlockSpec(memory_space=pl.ANY))
        io_alias = {6: 0}
    return pl.pallas_call(
        _tc_body,
        grid=(_BK,),
        in_specs=in_specs,
        out_specs=pl.BlockSpec((1, _SEQ, _HIDDEN),
                               lambda b, k=k: (b + k * _BK, 0, 0)),
        out_shape=jax.ShapeDtypeStruct((_BSZ, _SEQ, _HIDDEN), jnp.float32),
        input_output_aliases=io_alias,
    )(*ins)


def kernel(input_ids, token_type_ids, W_word, W_pos, W_type, gamma, beta):
    idx_flat = input_ids.reshape(-1).astype(jnp.int32)
    tt = token_type_ids.astype(jnp.int32).reshape(_BSZ, 1, _SEQ)
    w_type8 = jnp.zeros((8, _HIDDEN), jnp.float32).at[:2].set(W_type)
    gamma2d = gamma.reshape(1, _HIDDEN)
    beta2d = beta.reshape(1, _HIDDEN)

    gathered = [
        _sc_gather(W_word, idx_flat[k * _TOK_K:(k + 1) * _TOK_K])
        .reshape(_BK, _SEQ, _HIDDEN)
        for k in range(_K)
    ]
    y = None
    for k in range(_K):
        y = _tc_part(k, gathered[k], tt, w_pos=W_pos, w_type8=w_type8,
                     gamma2d=gamma2d, beta2d=beta2d, y_prev=y)
    return y


# speedup vs baseline: 3.3110x; 1.0019x over previous
import functools

import jax
import jax.numpy as jnp
from jax import lax
from jax.experimental import pallas as pl
from jax.experimental.pallas import tpu as pltpu
from jax.experimental.pallas import tpu_sc as plsc

_VOCAB = 100000
_HIDDEN = 768
_SEQ = 512
_BSZ = 128
_EPS = 1e-6

_B = _BSZ * _SEQ
_NC = 2
_NS = 16
_NW = _NC * _NS
_CHUNK = 64

_SLICES = (8, 24, 32, 32, 32)
_K = len(_SLICES)
_OFFS = tuple(sum(_SLICES[:i]) for i in range(_K))


def _make_sc_gather(n_batches):
    tok = n_batches * _SEQ
    b_per_w = tok // _NW
    n_chunks = b_per_w // _CHUNK

    def body(table_hbm, idx_hbm, out_hbm, idx_v, rows_v, sem0, sem1):
        wid = lax.axis_index("s") * _NC + lax.axis_index("c")
        base = wid * b_per_w
        pltpu.sync_copy(idx_hbm.at[pl.ds(base, b_per_w)], idx_v)
        sems = (sem0, sem1)

        def start(ci, b):
            pltpu.async_copy(
                table_hbm.at[idx_v.at[pl.ds(ci * _CHUNK, _CHUNK)]],
                rows_v.at[b], sems[b])

        start(0, 0)
        if n_chunks > 1:
            start(1, 1)
        for ci in range(n_chunks):
            b = ci % 2
            pltpu.make_async_copy(
                table_hbm.at[idx_v.at[pl.ds(ci * _CHUNK, _CHUNK)]],
                rows_v.at[b], sems[b]).wait()
            pltpu.sync_copy(rows_v.at[b],
                            out_hbm.at[pl.ds(base + ci * _CHUNK, _CHUNK)])
            if ci + 2 < n_chunks:
                start(ci + 2, b)

    return functools.partial(
        pl.kernel,
        mesh=plsc.VectorSubcoreMesh(core_axis_name="c", subcore_axis_name="s"),
        out_type=jax.ShapeDtypeStruct((tok, _HIDDEN), jnp.float32),
        scratch_types=[
            pltpu.VMEM((b_per_w,), jnp.int32),
            pltpu.VMEM((2, _CHUNK, _HIDDEN), jnp.float32),
            pltpu.SemaphoreType.DMA,
            pltpu.SemaphoreType.DMA,
        ],
    )(body)


_sc_gathers = {n: _make_sc_gather(n) for n in sorted(set(_SLICES))}


def _tc_body(x_ref, tt_ref, pos_ref, type_ref, gamma_ref, beta_ref, *rest):
    o_ref = rest[-1]
    x = x_ref[0]
    t = tt_ref[0, 0].astype(jnp.float32)
    pos = pos_ref[...]
    t0 = type_ref[0]
    dt = type_ref[1] - type_ref[0]
    x = x + pos + t0[None, :] + t[:, None] * dt[None, :]
    mean = jnp.mean(x, axis=-1, keepdims=True)
    xc = x - mean
    var = jnp.mean(xc * xc, axis=-1, keepdims=True)
    y = xc * lax.rsqrt(var + _EPS)
    o_ref[0] = y * gamma_ref[0][None, :] + beta_ref[0][None, :]


def _tc_part(k, gathered_k, tt, w_pos, w_type8, gamma2d, beta2d, y_prev):
    off = _OFFS[k]
    ins = [gathered_k, tt, w_pos, w_type8, gamma2d, beta2d]
    in_specs = [
        pl.BlockSpec((1, _SEQ, _HIDDEN), lambda b: (b, 0, 0)),
        pl.BlockSpec((1, 1, _SEQ), lambda b, off=off: (b + off, 0, 0)),
        pl.BlockSpec((_SEQ, _HIDDEN), lambda b: (0, 0)),
        pl.BlockSpec((8, _HIDDEN), lambda b: (0, 0)),
        pl.BlockSpec((1, _HIDDEN), lambda b: (0, 0)),
        pl.BlockSpec((1, _HIDDEN), lambda b: (0, 0)),
    ]
    io_alias = {}
    if y_prev is not None:
        ins.append(y_prev)
        in_specs.append(pl.BlockSpec(memory_space=pl.ANY))
        io_alias = {6: 0}
    return pl.pallas_call(
        _tc_body,
        grid=(_SLICES[k],),
        in_specs=in_specs,
        out_specs=pl.BlockSpec((1, _SEQ, _HIDDEN),
                               lambda b, off=off: (b + off, 0, 0)),
        out_shape=jax.ShapeDtypeStruct((_BSZ, _SEQ, _HIDDEN), jnp.float32),
        input_output_aliases=io_alias,
    )(*ins)


def kernel(input_ids, token_type_ids, W_word, W_pos, W_type, gamma, beta):
    idx_flat = input_ids.reshape(-1).astype(jnp.int32)
    tt = token_type_ids.astype(jnp.int32).reshape(_BSZ, 1, _SEQ)
    w_type8 = jnp.zeros((8, _HIDDEN), jnp.float32).at[:2].set(W_type)
    gamma2d = gamma.reshape(1, _HIDDEN)
    beta2d = beta.reshape(1, _HIDDEN)

    gathered = [
        _sc_gathers[_SLICES[k]](
            W_word,
            idx_flat[_OFFS[k] * _SEQ:(_OFFS[k] + _SLICES[k]) * _SEQ])
        .reshape(_SLICES[k], _SEQ, _HIDDEN)
        for k in range(_K)
    ]
    y = None
    for k in range(_K):
        y = _tc_part(k, gathered[k], tt, w_pos=W_pos, w_type8=w_type8,
                     gamma2d=gamma2d, beta2d=beta2d, y_prev=y)
    return y
